# Initial kernel scaffold; baseline (speedup 1.0000x reference)
import jax
import jax.numpy as jnp
from jax.experimental import pallas as pl


def kernel(input_ids, feature_ids, text_table, feature_table):
    raise NotImplementedError("write your pallas kernel here")


# speedup vs baseline: 1.7238x; 1.7238x over previous
import jax
import jax.numpy as jnp
from jax import lax
from jax.experimental import pallas as pl
from jax.experimental.pallas import tpu as pltpu, tpu_sc as plsc

B, S, H = 4, 8192, 1024
N = B * S
NC, NS = 2, 16
NW = NC * NS
PER_W = N // NW
CHUNK = 64
NCHUNK = PER_W // CHUNK


def _emb_body(text_table, feat_table, text_ids, feat_ids,
              text_out, feat_out, idx_v, rows_v, sem):
    wid = lax.axis_index("s") * NC + lax.axis_index("c")
    base = wid * PER_W
    pltpu.sync_copy(text_ids.at[pl.ds(base, PER_W)], idx_v.at[pl.ds(0, PER_W)])
    pltpu.sync_copy(feat_ids.at[pl.ds(base, PER_W)],
                    idx_v.at[pl.ds(PER_W, PER_W)])
    for t, (table, out) in enumerate(((text_table, text_out),
                                      (feat_table, feat_out))):
        def body(c, carry):
            off = c * CHUNK
            idx_slice = idx_v.at[pl.ds(t * PER_W + off, CHUNK)]
            pltpu.async_copy(table.at[idx_slice], rows_v, sem).wait()
            pltpu.sync_copy(rows_v, out.at[pl.ds(base + off, CHUNK)])
            return carry
        lax.fori_loop(0, NCHUNK, body, 0)


def kernel(input_ids, feature_ids, text_table, feature_table):
    t_ids = input_ids.reshape(-1).astype(jnp.int32)
    f_ids = feature_ids.reshape(-1).astype(jnp.int32)
    mesh = plsc.VectorSubcoreMesh(core_axis_name="c", subcore_axis_name="s")
    fn = pl.kernel(
        _emb_body,
        out_type=(jax.ShapeDtypeStruct((N, H), jnp.float32),
                  jax.ShapeDtypeStruct((N, H), jnp.float32)),
        mesh=mesh,
        scratch_types=[
            pltpu.VMEM((2 * PER_W,), jnp.int32),
            pltpu.VMEM((CHUNK, H), jnp.float32),
            pltpu.SemaphoreType.DMA,
        ],
    )
    t_out, f_out = fn(text_table, feature_table, t_ids, f_ids)
    return t_out.reshape(B, S, H), f_out.reshape(B, S, H)


# speedup vs baseline: 1.8713x; 1.0855x over previous
import jax
import jax.numpy as jnp
from jax import lax
from jax.experimental import pallas as pl
from jax.experimental.pallas import tpu as pltpu, tpu_sc as plsc

B, S, H = 4, 8192, 1024
N = B * S
NC, NS = 2, 16
NW = NC * NS
PER_W = N // NW
CHUNK = 32
NCHUNK = PER_W // CHUNK


def _emb_body(text_table, feat_table, text_ids, feat_ids,
              text_out, feat_out, idx_v, rows0, rows1, gsem0, gsem1):
    wid = lax.axis_index("s") * NC + lax.axis_index("c")
    base = wid * PER_W
    rows = (rows0, rows1)
    gsem = (gsem0, gsem1)
    pltpu.sync_copy(text_ids.at[pl.ds(base, PER_W)], idx_v.at[pl.ds(0, PER_W)])
    pltpu.sync_copy(feat_ids.at[pl.ds(base, PER_W)],
                    idx_v.at[pl.ds(PER_W, PER_W)])

    for t, (table, out) in enumerate(((text_table, text_out),
                                      (feat_table, feat_out))):
        def start_gather(ch, b):
            idx_s = idx_v.at[pl.ds(t * PER_W + ch * CHUNK, CHUNK)]
            pltpu.async_copy(table.at[idx_s], rows[b], gsem[b])

        def wait_gather(b):
            pltpu.make_async_copy(table.at[pl.ds(0, CHUNK)], rows[b],
                                  gsem[b]).wait()

        def scatter(ch, b):
            pltpu.sync_copy(rows[b], out.at[pl.ds(base + ch * CHUNK, CHUNK)])

        start_gather(0, 0)
        start_gather(1, 1)

        @pl.loop(0, NCHUNK - 2, step=2)
        def _(c):
            for b in range(2):
                ch = c + b
                wait_gather(b)
                scatter(ch, b)
                start_gather(ch + 2, b)

        for b in range(2):
            wait_gather(b)
            scatter(NCHUNK - 2 + b, b)


def kernel(input_ids, feature_ids, text_table, feature_table):
    t_ids = input_ids.reshape(-1).astype(jnp.int32)
    f_ids = feature_ids.reshape(-1).astype(jnp.int32)
    mesh = plsc.VectorSubcoreMesh(core_axis_name="c", subcore_axis_name="s")
    fn = pl.kernel(
        _emb_body,
        out_type=(jax.ShapeDtypeStruct((N, H), jnp.float32),
                  jax.ShapeDtypeStruct((N, H), jnp.float32)),
        mesh=mesh,
        scratch_types=[
            pltpu.VMEM((2 * PER_W,), jnp.int32),
            pltpu.VMEM((CHUNK, H), jnp.float32),
            pltpu.VMEM((CHUNK, H), jnp.float32),
            pltpu.SemaphoreType.DMA,
            pltpu.SemaphoreType.DMA,
        ],
    )
    t_out, f_out = fn(text_table, feature_table, t_ids, f_ids)
    return t_out.reshape(B, S, H), f_out.reshape(B, S, H)
